# baseline (speedup 1.0000x reference)
import functools

import jax
import jax.numpy as jnp
from jax import lax
from jax.experimental import pallas as pl
from jax.experimental.pallas import tpu as pltpu
from jax.experimental.pallas import tpu_sc as plsc

N_NODES = 10000
N_EDGES = 160000
F_IN = 256
F_HID = 512
F_OUT = 64

NC, NS = 2, 16
N_PAD = 10240
E_PAD = 163840
CH = 128
ROWS_PER_TILE = N_PAD // NS
F_HALF = F_IN // 2
DEG_W = 8

_mesh = plsc.VectorSubcoreMesh(core_axis_name="c", subcore_axis_name="s")
_sc_params = pltpu.CompilerParams(use_tc_tiling_on_sc=False)
_sc_params_nl = pltpu.CompilerParams(use_tc_tiling_on_sc=False,
                                     needs_layout_passes=False)

_CHUNKS_E = E_PAD // (NC * NS) // CH
_CHUNKS_F = E_PAD // NS // CH


_E_TILE = E_PAD // (NC * NS)


@functools.partial(
    pl.kernel,
    out_type=jax.ShapeDtypeStruct((NC * NS, N_PAD), jnp.float32),
    mesh=_mesh,
    scratch_types=[
        pltpu.VMEM((_E_TILE,), jnp.int32),
        pltpu.VMEM((N_PAD,), jnp.float32),
    ],
    compiler_params=_sc_params_nl,
)
def _deg_kernel(dst_hbm, out_hbm, idx_v, acc_v):
    c = lax.axis_index("c")
    s = lax.axis_index("s")
    wid = c * NS + s
    pltpu.sync_copy(dst_hbm.at[wid], idx_v)
    zeros = jnp.zeros((16,), jnp.float32)

    def zbody(i, carry):
        acc_v[pl.ds(i * 16, 16)] = zeros
        return carry

    lax.fori_loop(0, N_PAD // 16, zbody, 0)
    ones = jnp.ones((16,), jnp.float32)

    def body(i, carry):
        idx = idx_v[pl.ds(i * 16, 16)]
        plsc.addupdate_scatter(acc_v, [idx], ones)
        return carry

    lax.fori_loop(0, _E_TILE // 16, body, 0)
    pltpu.sync_copy(acc_v, out_hbm.at[wid])


@functools.partial(
    pl.kernel,
    out_type=(jax.ShapeDtypeStruct((N_PAD, F_HALF), jnp.float32),
              jax.ShapeDtypeStruct((N_PAD, F_HALF), jnp.float32)),
    mesh=_mesh,
    scratch_types=[
        pltpu.VMEM((_CHUNKS_F // 2, CH), jnp.int32),
        pltpu.VMEM((_CHUNKS_F // 2, CH), jnp.int32),
        pltpu.VMEM((CH, F_HALF), jnp.float32),
        pltpu.VMEM((CH, F_HALF), jnp.float32),
        pltpu.VMEM_SHARED((N_PAD, F_HALF), jnp.float32),
        pltpu.SemaphoreType.DMA,
        pltpu.SemaphoreType.DMA,
    ],
    compiler_params=_sc_params,
)
def _spmm_wide(xs_l, xs_r, src_hbm, dst_hbm, zeros_hbm, out_l, out_r,
               src_v, dst_v, buf0, buf1, acc, sem0, sem1):
    c = lax.axis_index("c")
    s = lax.axis_index("s")
    rsl = pl.ds(s * ROWS_PER_TILE, ROWS_PER_TILE)
    pltpu.sync_copy(zeros_hbm, acc.at[rsl])
    plsc.subcore_barrier()
    half = _CHUNKS_F // 2

    def pipe(table):
        for h in range(2):
            pltpu.sync_copy(src_hbm.at[s, pl.ds(h * half, half)], src_v)
            pltpu.sync_copy(dst_hbm.at[s, pl.ds(h * half, half)], dst_v)
            n2 = half // 2
            pltpu.async_copy(table.at[src_v.at[0]], buf0, sem0)

            def body(j2, carry):
                k0 = 2 * j2
                pltpu.make_async_copy(table.at[pl.ds(0, CH)], buf0,
                                      sem0).wait()
                pltpu.async_copy(table.at[src_v.at[k0 + 1]], buf1, sem1)
                pltpu.sync_copy(buf0, acc.at[dst_v.at[k0]], add=True)
                pltpu.make_async_copy(table.at[pl.ds(0, CH)], buf1,
                                      sem1).wait()

                @pl.when(j2 < n2 - 1)
                def _():
                    pltpu.async_copy(table.at[src_v.at[k0 + 2]], buf0, sem0)

                pltpu.sync_copy(buf1, acc.at[dst_v.at[k0 + 1]], add=True)
                return carry

            lax.fori_loop(0, n2, body, 0)

    @pl.when(c == 0)
    def _():
        pipe(xs_l)

    @pl.when(c == 1)
    def _():
        pipe(xs_r)

    plsc.subcore_barrier()

    @pl.when(c == 0)
    def _():
        pltpu.sync_copy(acc.at[rsl], out_l.at[rsl])

    @pl.when(c == 1)
    def _():
        pltpu.sync_copy(acc.at[rsl], out_r.at[rsl])


@functools.partial(
    pl.kernel,
    out_type=(jax.ShapeDtypeStruct((N_PAD, F_OUT), jnp.float32),
              jax.ShapeDtypeStruct((N_PAD, F_OUT), jnp.float32)),
    mesh=_mesh,
    scratch_types=[
        pltpu.VMEM((_CHUNKS_E, CH), jnp.int32),
        pltpu.VMEM((_CHUNKS_E, CH), jnp.int32),
        pltpu.VMEM((CH, F_OUT), jnp.float32),
        pltpu.VMEM((CH, F_OUT), jnp.float32),
        pltpu.VMEM_SHARED((N_PAD, F_OUT), jnp.float32),
        pltpu.SemaphoreType.DMA,
        pltpu.SemaphoreType.DMA,
    ],
    compiler_params=_sc_params,
)
def _spmm_narrow(p_hbm, src_hbm, dst_hbm, zeros_hbm, out_a, out_b,
                 src_v, dst_v, buf0, buf1, acc, sem0, sem1):
    c = lax.axis_index("c")
    s = lax.axis_index("s")
    wid = c * NS + s
    rsl = pl.ds(s * ROWS_PER_TILE, ROWS_PER_TILE)
    pltpu.sync_copy(src_hbm.at[wid], src_v)
    pltpu.sync_copy(dst_hbm.at[wid], dst_v)
    pltpu.sync_copy(zeros_hbm, acc.at[rsl])
    plsc.subcore_barrier()

    n2 = _CHUNKS_E // 2
    pltpu.async_copy(p_hbm.at[src_v.at[0]], buf0, sem0)

    def body(j2, carry):
        k0 = 2 * j2
        pltpu.make_async_copy(p_hbm.at[pl.ds(0, CH)], buf0, sem0).wait()
        pltpu.async_copy(p_hbm.at[src_v.at[k0 + 1]], buf1, sem1)
        pltpu.sync_copy(buf0, acc.at[dst_v.at[k0]], add=True)
        pltpu.make_async_copy(p_hbm.at[pl.ds(0, CH)], buf1, sem1).wait()

        @pl.when(j2 < n2 - 1)
        def _():
            pltpu.async_copy(p_hbm.at[src_v.at[k0 + 2]], buf0, sem0)

        pltpu.sync_copy(buf1, acc.at[dst_v.at[k0 + 1]], add=True)
        return carry

    lax.fori_loop(0, n2, body, 0)
    plsc.subcore_barrier()

    @pl.when(c == 0)
    def _():
        pltpu.sync_copy(acc.at[rsl], out_a.at[rsl])

    @pl.when(c == 1)
    def _():
        pltpu.sync_copy(acc.at[rsl], out_b.at[rsl])


_RB = 1280


def _prescale_body(deg_ref, x_ref, a_ref, xs_l_ref, xs_r_ref):
    deg = jnp.sum(deg_ref[...], axis=0)
    a = 1.0 / jnp.sqrt(jnp.maximum(deg, 1.0))
    a2 = a[:, None]
    a_ref[...] = a2
    xs = x_ref[...] * a2
    xs_l_ref[...] = xs[:, :F_HALF]
    xs_r_ref[...] = xs[:, F_HALF:]


def _dense_body(v1l_ref, v1r_ref, a_ref, w1a_ref, w1b_ref, b1_ref, w2_ref,
                p_ref):
    a2 = a_ref[...]
    hp = jnp.dot(v1l_ref[...] * a2, w1a_ref[...],
                 preferred_element_type=jnp.float32,
                 precision=lax.Precision.HIGHEST)
    hp += jnp.dot(v1r_ref[...] * a2, w1b_ref[...],
                  preferred_element_type=jnp.float32,
                  precision=lax.Precision.HIGHEST)
    h = jnp.maximum(hp + b1_ref[...], 0.0)
    p_ref[...] = jnp.dot(h, w2_ref[...],
                         preferred_element_type=jnp.float32,
                         precision=lax.Precision.HIGHEST) * a2


def _finish_body(va_ref, vb_ref, a_ref, b2_ref, out_ref):
    out_ref[...] = (va_ref[...] + vb_ref[...]) * a_ref[...] + b2_ref[...]


def _row_spec(width):
    return pl.BlockSpec((_RB, width), lambda i: (i, 0))


def _full_spec(shape):
    nd = len(shape)
    return pl.BlockSpec(shape, lambda i, _n=nd: (0,) * _n)


_prescale = pl.pallas_call(
    _prescale_body,
    grid=(N_PAD // _RB,),
    in_specs=[pl.BlockSpec((NC * NS, _RB), lambda i: (0, i)),
              _row_spec(F_IN)],
    out_specs=(_row_spec(1), _row_spec(F_HALF), _row_spec(F_HALF)),
    out_shape=(jax.ShapeDtypeStruct((N_PAD, 1), jnp.float32),
               jax.ShapeDtypeStruct((N_PAD, F_HALF), jnp.float32),
               jax.ShapeDtypeStruct((N_PAD, F_HALF), jnp.float32)),
)

_dense = pl.pallas_call(
    _dense_body,
    grid=(N_PAD // _RB,),
    in_specs=[_row_spec(F_HALF), _row_spec(F_HALF), _row_spec(1),
              _full_spec((F_HALF, F_HID)), _full_spec((F_HALF, F_HID)),
              _full_spec((1, F_HID)), _full_spec((F_HID, F_OUT))],
    out_specs=_row_spec(F_OUT),
    out_shape=jax.ShapeDtypeStruct((N_PAD, F_OUT), jnp.float32),
)

_finish = pl.pallas_call(
    _finish_body,
    grid=(N_PAD // _RB,),
    in_specs=[_row_spec(F_OUT), _row_spec(F_OUT), _row_spec(1),
              _full_spec((1, F_OUT))],
    out_specs=_row_spec(F_OUT),
    out_shape=jax.ShapeDtypeStruct((N_PAD, F_OUT), jnp.float32),
)


def kernel(x, edge_index, edge_weight, W1, b1, W2, b2):
    del edge_weight
    src = edge_index[0]
    dst = edge_index[1]
    pad = E_PAD - N_EDGES
    src_p = jnp.concatenate([src, jnp.full((pad,), N_NODES, jnp.int32)])
    dst_p = jnp.concatenate([dst, jnp.full((pad,), N_NODES, jnp.int32)])
    src32 = src_p.reshape(NC * NS, _CHUNKS_E, CH)
    dst32 = dst_p.reshape(NC * NS, _CHUNKS_E, CH)
    src16 = src_p.reshape(NS, _CHUNKS_F, CH)
    dst16 = dst_p.reshape(NS, _CHUNKS_F, CH)
    x_pad = jnp.zeros((N_PAD, F_IN), jnp.float32).at[:N_NODES].set(x)

    z_wide = jnp.zeros((ROWS_PER_TILE, F_HALF), jnp.float32)
    z_nar = jnp.zeros((ROWS_PER_TILE, F_OUT), jnp.float32)

    deg_parts = _deg_kernel(dst_p.reshape(NC * NS, _E_TILE))
    a, xs_l, xs_r = _prescale(deg_parts, x_pad)
    v1_l, v1_r = _spmm_wide(xs_l, xs_r, src16, dst16, z_wide)
    p = _dense(v1_l, v1_r, a, W1[:F_HALF], W1[F_HALF:],
               b1.reshape(1, F_HID), W2)
    v2a, v2b = _spmm_narrow(p, src32, dst32, z_nar)
    out = _finish(v2a, v2b, a, b2.reshape(1, F_OUT))
    return out[:N_NODES]


# speedup vs baseline: 1.0322x; 1.0322x over previous
import functools

import jax
import jax.numpy as jnp
from jax import lax
from jax.experimental import pallas as pl
from jax.experimental.pallas import tpu as pltpu
from jax.experimental.pallas import tpu_sc as plsc

N_NODES = 10000
N_EDGES = 160000
F_IN = 256
F_HID = 512
F_OUT = 64

NC, NS = 2, 16
N_PAD = 10240
E_PAD = 163840
CH = 128
ROWS_PER_TILE = N_PAD // NS
F_HALF = F_IN // 2
DEG_W = 8

_mesh = plsc.VectorSubcoreMesh(core_axis_name="c", subcore_axis_name="s")
_sc_params = pltpu.CompilerParams(use_tc_tiling_on_sc=False)
_sc_params_nl = pltpu.CompilerParams(use_tc_tiling_on_sc=False,
                                     needs_layout_passes=False)

_CHUNKS_E = E_PAD // (NC * NS) // CH
_CHUNKS_F = E_PAD // NS // CH


_E_TILE = E_PAD // (NC * NS)


@functools.partial(
    pl.kernel,
    out_type=jax.ShapeDtypeStruct((NC * NS, N_PAD), jnp.float32),
    mesh=_mesh,
    scratch_types=[
        pltpu.VMEM((_E_TILE,), jnp.int32),
        pltpu.VMEM((N_PAD,), jnp.float32),
    ],
    compiler_params=_sc_params_nl,
)
def _deg_kernel(dst_hbm, out_hbm, idx_v, acc_v):
    c = lax.axis_index("c")
    s = lax.axis_index("s")
    wid = c * NS + s
    pltpu.sync_copy(dst_hbm.at[wid], idx_v)
    zeros = jnp.zeros((16,), jnp.float32)

    def zbody(i, carry):
        acc_v[pl.ds(i * 16, 16)] = zeros
        return carry

    lax.fori_loop(0, N_PAD // 16, zbody, 0)
    ones = jnp.ones((16,), jnp.float32)

    def body(i, carry):
        idx = idx_v[pl.ds(i * 16, 16)]
        plsc.addupdate_scatter(acc_v, [idx], ones)
        return carry

    lax.fori_loop(0, _E_TILE // 16, body, 0)
    pltpu.sync_copy(acc_v, out_hbm.at[wid])


@functools.partial(
    pl.kernel,
    out_type=(jax.ShapeDtypeStruct((N_PAD, F_HALF), jnp.float32),
              jax.ShapeDtypeStruct((N_PAD, F_HALF), jnp.float32)),
    mesh=_mesh,
    scratch_types=[
        pltpu.VMEM((_CHUNKS_F, CH), jnp.int32),
        pltpu.VMEM((_CHUNKS_F, CH), jnp.int32),
        pltpu.VMEM((CH, F_HALF), jnp.float32),
        pltpu.VMEM_SHARED((N_PAD, F_HALF), jnp.float32),
        pltpu.SemaphoreType.DMA,
    ],
    compiler_params=_sc_params,
)
def _spmm_wide(xs_l, xs_r, src_hbm, dst_hbm, zeros_hbm, out_l, out_r,
               src_v, dst_v, rows_v, acc, sem):
    c = lax.axis_index("c")
    s = lax.axis_index("s")
    rsl = pl.ds(s * ROWS_PER_TILE, ROWS_PER_TILE)
    pltpu.sync_copy(src_hbm.at[s], src_v)
    pltpu.sync_copy(dst_hbm.at[s], dst_v)
    pltpu.sync_copy(zeros_hbm, acc.at[rsl])
    plsc.subcore_barrier()

    def pipe(table):
        def body(j, carry):
            pltpu.async_copy(table.at[src_v.at[j]], rows_v, sem).wait()
            pltpu.sync_copy(rows_v, acc.at[dst_v.at[j]], add=True)
            return carry

        lax.fori_loop(0, _CHUNKS_F, body, 0)

    @pl.when(c == 0)
    def _():
        pipe(xs_l)

    @pl.when(c == 1)
    def _():
        pipe(xs_r)

    plsc.subcore_barrier()

    @pl.when(c == 0)
    def _():
        pltpu.sync_copy(acc.at[rsl], out_l.at[rsl])

    @pl.when(c == 1)
    def _():
        pltpu.sync_copy(acc.at[rsl], out_r.at[rsl])


F_Q = F_OUT // 2


@functools.partial(
    pl.kernel,
    out_type=(jax.ShapeDtypeStruct((N_PAD, F_Q), jnp.float32),
              jax.ShapeDtypeStruct((N_PAD, F_Q), jnp.float32)),
    mesh=_mesh,
    scratch_types=[
        pltpu.VMEM((_CHUNKS_F, CH), jnp.int32),
        pltpu.VMEM((_CHUNKS_F, CH), jnp.int32),
        pltpu.VMEM((CH, F_Q), jnp.float32),
        pltpu.VMEM((ROWS_PER_TILE, F_Q), jnp.float32),
        pltpu.VMEM((ROWS_PER_TILE, 16), jnp.float32),
        pltpu.VMEM((F_Q,), jnp.float32),
        pltpu.VMEM_SHARED((N_PAD, F_Q), jnp.float32),
        pltpu.SemaphoreType.DMA,
    ],
    compiler_params=_sc_params,
)
def _spmm_narrow(p_l, p_r, src_hbm, dst_hbm, zeros_hbm, a_hbm, b2_hbm,
                 out_l, out_r, src_v, dst_v, rows_v, tmp_v, a_v, b2_v,
                 acc, sem):
    c = lax.axis_index("c")
    s = lax.axis_index("s")
    rsl = pl.ds(s * ROWS_PER_TILE, ROWS_PER_TILE)
    pltpu.sync_copy(src_hbm.at[s], src_v)
    pltpu.sync_copy(dst_hbm.at[s], dst_v)
    pltpu.sync_copy(zeros_hbm, acc.at[rsl])
    pltpu.sync_copy(a_hbm.at[rsl], a_v)
    pltpu.sync_copy(b2_hbm.at[pl.ds(c * F_Q, F_Q)], b2_v)
    plsc.subcore_barrier()

    def pipe(table):
        def body(j, carry):
            pltpu.async_copy(table.at[src_v.at[j]], rows_v, sem).wait()
            pltpu.sync_copy(rows_v, acc.at[dst_v.at[j]], add=True)
            return carry

        lax.fori_loop(0, _CHUNKS_F, body, 0)

    @pl.when(c == 0)
    def _():
        pipe(p_l)

    @pl.when(c == 1)
    def _():
        pipe(p_r)

    plsc.subcore_barrier()
    pltpu.sync_copy(acc.at[rsl], tmp_v)
    b2a = b2_v[pl.ds(0, 16)]
    b2b = b2_v[pl.ds(16, 16)]

    def rbody(i, carry):
        av = a_v[i, pl.ds(0, 16)]
        tmp_v[i, pl.ds(0, 16)] = tmp_v[i, pl.ds(0, 16)] * av + b2a
        tmp_v[i, pl.ds(16, 16)] = tmp_v[i, pl.ds(16, 16)] * av + b2b
        return carry

    lax.fori_loop(0, ROWS_PER_TILE, rbody, 0)

    @pl.when(c == 0)
    def _():
        pltpu.sync_copy(tmp_v, out_l.at[rsl])

    @pl.when(c == 1)
    def _():
        pltpu.sync_copy(tmp_v, out_r.at[rsl])


_RB = 1280


def _prescale_body(deg_ref, x_ref, a_ref, xs_l_ref, xs_r_ref):
    deg = jnp.sum(deg_ref[...], axis=0)
    a = 1.0 / jnp.sqrt(jnp.maximum(deg, 1.0))
    a2 = a[:, None]
    a_ref[...] = a2
    xs = x_ref[...] * a2
    xs_l_ref[...] = xs[:, :F_HALF]
    xs_r_ref[...] = xs[:, F_HALF:]


def _dense_body(v1l_ref, v1r_ref, a_ref, w1a_ref, w1b_ref, b1_ref, w2_ref,
                pl_ref, pr_ref):
    a2 = a_ref[...]
    hp = jnp.dot(v1l_ref[...] * a2, w1a_ref[...],
                 preferred_element_type=jnp.float32,
                 precision=lax.Precision.HIGHEST)
    hp += jnp.dot(v1r_ref[...] * a2, w1b_ref[...],
                  preferred_element_type=jnp.float32,
                  precision=lax.Precision.HIGHEST)
    h = jnp.maximum(hp + b1_ref[...], 0.0)
    p = jnp.dot(h, w2_ref[...],
                preferred_element_type=jnp.float32,
                precision=lax.Precision.HIGHEST) * a2
    pl_ref[...] = p[:, :F_Q]
    pr_ref[...] = p[:, F_Q:]


def _row_spec(width):
    return pl.BlockSpec((_RB, width), lambda i: (i, 0))


def _full_spec(shape):
    nd = len(shape)
    return pl.BlockSpec(shape, lambda i, _n=nd: (0,) * _n)


_prescale = pl.pallas_call(
    _prescale_body,
    grid=(N_PAD // _RB,),
    in_specs=[pl.BlockSpec((NC * NS, _RB), lambda i: (0, i)),
              _row_spec(F_IN)],
    out_specs=(_row_spec(1), _row_spec(F_HALF), _row_spec(F_HALF)),
    out_shape=(jax.ShapeDtypeStruct((N_PAD, 1), jnp.float32),
               jax.ShapeDtypeStruct((N_PAD, F_HALF), jnp.float32),
               jax.ShapeDtypeStruct((N_PAD, F_HALF), jnp.float32)),
)

_dense = pl.pallas_call(
    _dense_body,
    grid=(N_PAD // _RB,),
    in_specs=[_row_spec(F_HALF), _row_spec(F_HALF), _row_spec(1),
              _full_spec((F_HALF, F_HID)), _full_spec((F_HALF, F_HID)),
              _full_spec((1, F_HID)), _full_spec((F_HID, F_OUT))],
    out_specs=(_row_spec(F_Q), _row_spec(F_Q)),
    out_shape=(jax.ShapeDtypeStruct((N_PAD, F_Q), jnp.float32),
               jax.ShapeDtypeStruct((N_PAD, F_Q), jnp.float32)),
)


def kernel(x, edge_index, edge_weight, W1, b1, W2, b2):
    del edge_weight
    src = edge_index[0]
    dst = edge_index[1]
    pad = E_PAD - N_EDGES
    src_p = jnp.concatenate([src, jnp.full((pad,), N_NODES, jnp.int32)])
    dst_p = jnp.concatenate([dst, jnp.full((pad,), N_NODES, jnp.int32)])
    src16 = src_p.reshape(NS, _CHUNKS_F, CH)
    dst16 = dst_p.reshape(NS, _CHUNKS_F, CH)
    x_pad = jnp.zeros((N_PAD, F_IN), jnp.float32).at[:N_NODES].set(x)

    z_wide = jnp.zeros((ROWS_PER_TILE, F_HALF), jnp.float32)
    z_nar = jnp.zeros((ROWS_PER_TILE, F_Q), jnp.float32)

    deg_parts = _deg_kernel(dst_p.reshape(NC * NS, _E_TILE))
    a, xs_l, xs_r = _prescale(deg_parts, x_pad)
    v1_l, v1_r = _spmm_wide(xs_l, xs_r, src16, dst16, z_wide)
    p_l, p_r = _dense(v1_l, v1_r, a, W1[:F_HALF], W1[F_HALF:],
                      b1.reshape(1, F_HID), W2)
    o_l, o_r = _spmm_narrow(p_l, p_r, src16, dst16, z_nar,
                            jnp.broadcast_to(a, (N_PAD, 16)), b2)
    return jnp.concatenate([o_l, o_r], axis=1)[:N_NODES]


# speedup vs baseline: 1.1735x; 1.1369x over previous
import functools

import jax
import jax.numpy as jnp
from jax import lax
from jax.experimental import pallas as pl
from jax.experimental.pallas import tpu as pltpu
from jax.experimental.pallas import tpu_sc as plsc

N_NODES = 10000
N_EDGES = 160000
F_IN = 256
F_HID = 512
F_OUT = 64

NC, NS = 2, 16
N_PAD = 10240
E_PAD = 163840
CH = 128
ROWS_PER_TILE = N_PAD // NS
F_HALF = F_IN // 2
DEG_W = 8

_mesh = plsc.VectorSubcoreMesh(core_axis_name="c", subcore_axis_name="s")
_sc_params = pltpu.CompilerParams(use_tc_tiling_on_sc=False)
_sc_params_nl = pltpu.CompilerParams(use_tc_tiling_on_sc=False,
                                     needs_layout_passes=False)

_CHUNKS_E = E_PAD // (NC * NS) // CH
_CHUNKS_F = E_PAD // NS // CH


_E_TILE = E_PAD // (NC * NS)


@functools.partial(
    pl.kernel,
    out_type=jax.ShapeDtypeStruct((NC * NS, N_PAD), jnp.float32),
    mesh=_mesh,
    scratch_types=[
        pltpu.VMEM((_E_TILE,), jnp.int32),
        pltpu.VMEM((N_PAD,), jnp.float32),
    ],
    compiler_params=_sc_params_nl,
)
def _deg_kernel(dst_hbm, out_hbm, idx_v, acc_v):
    c = lax.axis_index("c")
    s = lax.axis_index("s")
    wid = c * NS + s
    pltpu.sync_copy(dst_hbm.at[wid], idx_v)
    zeros = jnp.zeros((16,), jnp.float32)

    def zbody(i, carry):
        acc_v[pl.ds(i * 16, 16)] = zeros
        return carry

    lax.fori_loop(0, N_PAD // 16, zbody, 0)
    ones = jnp.ones((16,), jnp.float32)

    def body(i, carry):
        idx = idx_v[pl.ds(i * 16, 16)]
        plsc.addupdate_scatter(acc_v, [idx], ones)
        return carry

    lax.fori_loop(0, _E_TILE // 16, body, 0)
    pltpu.sync_copy(acc_v, out_hbm.at[wid])


@functools.partial(
    pl.kernel,
    out_type=(jax.ShapeDtypeStruct((N_PAD, F_HALF), jnp.float32),
              jax.ShapeDtypeStruct((N_PAD, F_HALF), jnp.float32)),
    mesh=_mesh,
    scratch_types=[
        pltpu.VMEM((_CHUNKS_F, CH), jnp.int32),
        pltpu.VMEM((_CHUNKS_F, CH), jnp.int32),
        pltpu.VMEM((CH, F_HALF), jnp.float32),
        pltpu.VMEM_SHARED((N_PAD, F_HALF), jnp.float32),
        pltpu.SemaphoreType.DMA,
    ],
    compiler_params=_sc_params,
)
def _spmm_wide(xs_l, xs_r, src_hbm, dst_hbm, zeros_hbm, out_l, out_r,
               src_v, dst_v, rows_v, acc, sem):
    c = lax.axis_index("c")
    s = lax.axis_index("s")
    rsl = pl.ds(s * ROWS_PER_TILE, ROWS_PER_TILE)
    pltpu.sync_copy(src_hbm.at[s], src_v)
    pltpu.sync_copy(dst_hbm.at[s], dst_v)
    pltpu.sync_copy(zeros_hbm, acc.at[rsl])
    plsc.subcore_barrier()

    def pipe(table):
        def body(j, carry):
            pltpu.async_copy(table.at[src_v.at[j]], rows_v, sem).wait()
            pltpu.sync_copy(rows_v, acc.at[dst_v.at[j]], add=True)
            return carry

        lax.fori_loop(0, _CHUNKS_F, body, 0)

    @pl.when(c == 0)
    def _():
        pipe(xs_l)

    @pl.when(c == 1)
    def _():
        pipe(xs_r)

    plsc.subcore_barrier()

    @pl.when(c == 0)
    def _():
        pltpu.sync_copy(acc.at[rsl], out_l.at[rsl])

    @pl.when(c == 1)
    def _():
        pltpu.sync_copy(acc.at[rsl], out_r.at[rsl])


F_Q = F_OUT // 2


@functools.partial(
    pl.kernel,
    out_type=(jax.ShapeDtypeStruct((N_PAD, F_Q), jnp.float32),
              jax.ShapeDtypeStruct((N_PAD, F_Q), jnp.float32)),
    mesh=_mesh,
    scratch_types=[
        pltpu.VMEM((_CHUNKS_F, CH), jnp.int32),
        pltpu.VMEM((_CHUNKS_F, CH), jnp.int32),
        pltpu.VMEM((CH, F_Q), jnp.float32),
        pltpu.VMEM((ROWS_PER_TILE, F_Q), jnp.float32),
        pltpu.VMEM((ROWS_PER_TILE, 16), jnp.float32),
        pltpu.VMEM((F_Q,), jnp.float32),
        pltpu.VMEM_SHARED((N_PAD, F_Q), jnp.float32),
        pltpu.VMEM_SHARED((N_PAD, F_Q), jnp.float32),
        pltpu.SemaphoreType.DMA,
    ],
    compiler_params=_sc_params,
)
def _spmm_narrow(p_l, p_r, src_hbm, dst_hbm, zeros_hbm, a_hbm, b2_hbm,
                 out_l, out_r, src_v, dst_v, rows_v, tmp_v, a_v, b2_v,
                 acc, table_sp, sem):
    c = lax.axis_index("c")
    s = lax.axis_index("s")
    rsl = pl.ds(s * ROWS_PER_TILE, ROWS_PER_TILE)
    pltpu.sync_copy(src_hbm.at[s], src_v)
    pltpu.sync_copy(dst_hbm.at[s], dst_v)
    pltpu.sync_copy(zeros_hbm, acc.at[rsl])
    pltpu.sync_copy(a_hbm.at[rsl], a_v)
    pltpu.sync_copy(b2_hbm.at[pl.ds(c * F_Q, F_Q)], b2_v)

    @pl.when(c == 0)
    def _():
        pltpu.sync_copy(p_l.at[rsl], table_sp.at[rsl])

    @pl.when(c == 1)
    def _():
        pltpu.sync_copy(p_r.at[rsl], table_sp.at[rsl])

    plsc.subcore_barrier()

    def body(j, carry):
        pltpu.async_copy(table_sp.at[src_v.at[j]], rows_v, sem).wait()
        pltpu.sync_copy(rows_v, acc.at[dst_v.at[j]], add=True)
        return carry

    lax.fori_loop(0, _CHUNKS_F, body, 0)
    plsc.subcore_barrier()
    pltpu.sync_copy(acc.at[rsl], tmp_v)
    b2a = b2_v[pl.ds(0, 16)]
    b2b = b2_v[pl.ds(16, 16)]

    def rbody(i, carry):
        av = a_v[i, pl.ds(0, 16)]
        tmp_v[i, pl.ds(0, 16)] = tmp_v[i, pl.ds(0, 16)] * av + b2a
        tmp_v[i, pl.ds(16, 16)] = tmp_v[i, pl.ds(16, 16)] * av + b2b
        return carry

    lax.fori_loop(0, ROWS_PER_TILE, rbody, 0)

    @pl.when(c == 0)
    def _():
        pltpu.sync_copy(tmp_v, out_l.at[rsl])

    @pl.when(c == 1)
    def _():
        pltpu.sync_copy(tmp_v, out_r.at[rsl])


_RB = 1280


def _prescale_body(deg_ref, x_ref, a_ref, xs_l_ref, xs_r_ref):
    deg = jnp.sum(deg_ref[...], axis=0)
    a = 1.0 / jnp.sqrt(jnp.maximum(deg, 1.0))
    a2 = a[:, None]
    a_ref[...] = a2
    xs = x_ref[...] * a2
    xs_l_ref[...] = xs[:, :F_HALF]
    xs_r_ref[...] = xs[:, F_HALF:]


def _dense_body(v1l_ref, v1r_ref, a_ref, w1a_ref, w1b_ref, b1_ref, w2_ref,
                pl_ref, pr_ref):
    a2 = a_ref[...]
    hp = jnp.dot(v1l_ref[...] * a2, w1a_ref[...],
                 preferred_element_type=jnp.float32,
                 precision=lax.Precision.HIGHEST)
    hp += jnp.dot(v1r_ref[...] * a2, w1b_ref[...],
                  preferred_element_type=jnp.float32,
                  precision=lax.Precision.HIGHEST)
    h = jnp.maximum(hp + b1_ref[...], 0.0)
    p = jnp.dot(h, w2_ref[...],
                preferred_element_type=jnp.float32,
                precision=lax.Precision.HIGHEST) * a2
    pl_ref[...] = p[:, :F_Q]
    pr_ref[...] = p[:, F_Q:]


def _row_spec(width):
    return pl.BlockSpec((_RB, width), lambda i: (i, 0))


def _full_spec(shape):
    nd = len(shape)
    return pl.BlockSpec(shape, lambda i, _n=nd: (0,) * _n)


_prescale = pl.pallas_call(
    _prescale_body,
    grid=(N_PAD // _RB,),
    in_specs=[pl.BlockSpec((NC * NS, _RB), lambda i: (0, i)),
              _row_spec(F_IN)],
    out_specs=(_row_spec(1), _row_spec(F_HALF), _row_spec(F_HALF)),
    out_shape=(jax.ShapeDtypeStruct((N_PAD, 1), jnp.float32),
               jax.ShapeDtypeStruct((N_PAD, F_HALF), jnp.float32),
               jax.ShapeDtypeStruct((N_PAD, F_HALF), jnp.float32)),
)

_dense = pl.pallas_call(
    _dense_body,
    grid=(N_PAD // _RB,),
    in_specs=[_row_spec(F_HALF), _row_spec(F_HALF), _row_spec(1),
              _full_spec((F_HALF, F_HID)), _full_spec((F_HALF, F_HID)),
              _full_spec((1, F_HID)), _full_spec((F_HID, F_OUT))],
    out_specs=(_row_spec(F_Q), _row_spec(F_Q)),
    out_shape=(jax.ShapeDtypeStruct((N_PAD, F_Q), jnp.float32),
               jax.ShapeDtypeStruct((N_PAD, F_Q), jnp.float32)),
)


def kernel(x, edge_index, edge_weight, W1, b1, W2, b2):
    del edge_weight
    src = edge_index[0]
    dst = edge_index[1]
    pad = E_PAD - N_EDGES
    src_p = jnp.concatenate([src, jnp.full((pad,), N_NODES, jnp.int32)])
    dst_p = jnp.concatenate([dst, jnp.full((pad,), N_NODES, jnp.int32)])
    src16 = src_p.reshape(NS, _CHUNKS_F, CH)
    dst16 = dst_p.reshape(NS, _CHUNKS_F, CH)
    x_pad = jnp.zeros((N_PAD, F_IN), jnp.float32).at[:N_NODES].set(x)

    z_wide = jnp.zeros((ROWS_PER_TILE, F_HALF), jnp.float32)
    z_nar = jnp.zeros((ROWS_PER_TILE, F_Q), jnp.float32)

    deg_parts = _deg_kernel(dst_p.reshape(NC * NS, _E_TILE))
    a, xs_l, xs_r = _prescale(deg_parts, x_pad)
    v1_l, v1_r = _spmm_wide(xs_l, xs_r, src16, dst16, z_wide)
    p_l, p_r = _dense(v1_l, v1_r, a, W1[:F_HALF], W1[F_HALF:],
                      b1.reshape(1, F_HID), W2)
    o_l, o_r = _spmm_narrow(p_l, p_r, src16, dst16, z_nar,
                            jnp.broadcast_to(a, (N_PAD, 16)), b2)
    return jnp.concatenate([o_l, o_r], axis=1)[:N_NODES]


# speedup vs baseline: 1.4635x; 1.2471x over previous
import functools

import jax
import jax.numpy as jnp
from jax import lax
from jax.experimental import pallas as pl
from jax.experimental.pallas import tpu as pltpu
from jax.experimental.pallas import tpu_sc as plsc

N_NODES = 10000
N_EDGES = 160000
F_IN = 256
F_HID = 512
F_OUT = 64

NC, NS = 2, 16
N_PAD = 10240
E_PAD = 163840
CH = 128
ROWS_PER_TILE = N_PAD // NS
F_HALF = F_IN // 2
DEG_W = 8

_mesh = plsc.VectorSubcoreMesh(core_axis_name="c", subcore_axis_name="s")
_sc_params = pltpu.CompilerParams(use_tc_tiling_on_sc=False)
_sc_params_nl = pltpu.CompilerParams(use_tc_tiling_on_sc=False,
                                     needs_layout_passes=False)

_CHUNKS_E = E_PAD // (NC * NS) // CH
_CHUNKS_F = E_PAD // NS // CH


_E_TILE = E_PAD // (NC * NS)


@functools.partial(
    pl.kernel,
    out_type=jax.ShapeDtypeStruct((NC * NS, N_PAD), jnp.float32),
    mesh=_mesh,
    scratch_types=[
        pltpu.VMEM((_E_TILE,), jnp.int32),
        pltpu.VMEM((N_PAD,), jnp.float32),
    ],
    compiler_params=_sc_params_nl,
)
def _deg_kernel(dst_hbm, out_hbm, idx_v, acc_v):
    c = lax.axis_index("c")
    s = lax.axis_index("s")
    wid = c * NS + s
    pltpu.sync_copy(dst_hbm.at[wid], idx_v)
    zeros = jnp.zeros((16,), jnp.float32)

    def zbody(i, carry):
        acc_v[pl.ds(i * 16, 16)] = zeros
        return carry

    lax.fori_loop(0, N_PAD // 16, zbody, 0)
    ones = jnp.ones((16,), jnp.float32)

    def body(i, carry):
        idx = idx_v[pl.ds(i * 16, 16)]
        plsc.addupdate_scatter(acc_v, [idx], ones)
        return carry

    lax.fori_loop(0, _E_TILE // 16, body, 0)
    pltpu.sync_copy(acc_v, out_hbm.at[wid])


F_Q4 = F_IN // 4


@functools.partial(
    pl.kernel,
    out_type=tuple(jax.ShapeDtypeStruct((N_PAD, F_Q4), jnp.float32)
                   for _ in range(4)),
    mesh=_mesh,
    scratch_types=[
        pltpu.VMEM((_CHUNKS_F, CH), jnp.int32),
        pltpu.VMEM((_CHUNKS_F, CH), jnp.int32),
        pltpu.VMEM((CH, F_Q4), jnp.float32),
        pltpu.VMEM_SHARED((N_PAD, F_Q4), jnp.float32),
        pltpu.VMEM_SHARED((N_PAD, F_Q4), jnp.float32),
        pltpu.SemaphoreType.DMA,
    ],
    compiler_params=_sc_params,
)
def _spmm_wide(xs_q0, xs_q1, xs_q2, xs_q3, src_hbm, dst_hbm, zeros_hbm,
               out_q0, out_q1, out_q2, out_q3,
               src_v, dst_v, rows_v, acc, table_sp, sem):
    c = lax.axis_index("c")
    s = lax.axis_index("s")
    rsl = pl.ds(s * ROWS_PER_TILE, ROWS_PER_TILE)
    pltpu.sync_copy(src_hbm.at[s], src_v)
    pltpu.sync_copy(dst_hbm.at[s], dst_v)

    def one_pass(tbl, out_q):
        pltpu.sync_copy(tbl.at[rsl], table_sp.at[rsl])
        pltpu.sync_copy(zeros_hbm, acc.at[rsl])
        plsc.subcore_barrier()

        def body(j, carry):
            pltpu.async_copy(table_sp.at[src_v.at[j]], rows_v, sem).wait()
            pltpu.sync_copy(rows_v, acc.at[dst_v.at[j]], add=True)
            return carry

        lax.fori_loop(0, _CHUNKS_F, body, 0)
        plsc.subcore_barrier()
        pltpu.sync_copy(acc.at[rsl], out_q.at[rsl])
        plsc.subcore_barrier()

    @pl.when(c == 0)
    def _():
        one_pass(xs_q0, out_q0)
        one_pass(xs_q1, out_q1)

    @pl.when(c == 1)
    def _():
        one_pass(xs_q2, out_q2)
        one_pass(xs_q3, out_q3)


F_Q = F_OUT // 2


@functools.partial(
    pl.kernel,
    out_type=(jax.ShapeDtypeStruct((N_PAD, F_Q), jnp.float32),
              jax.ShapeDtypeStruct((N_PAD, F_Q), jnp.float32)),
    mesh=_mesh,
    scratch_types=[
        pltpu.VMEM((_CHUNKS_F, CH), jnp.int32),
        pltpu.VMEM((_CHUNKS_F, CH), jnp.int32),
        pltpu.VMEM((CH, F_Q), jnp.float32),
        pltpu.VMEM((ROWS_PER_TILE, F_Q), jnp.float32),
        pltpu.VMEM((ROWS_PER_TILE, 16), jnp.float32),
        pltpu.VMEM((F_Q,), jnp.float32),
        pltpu.VMEM_SHARED((N_PAD, F_Q), jnp.float32),
        pltpu.VMEM_SHARED((N_PAD, F_Q), jnp.float32),
        pltpu.SemaphoreType.DMA,
    ],
    compiler_params=_sc_params,
)
def _spmm_narrow(p_l, p_r, src_hbm, dst_hbm, zeros_hbm, a_hbm, b2_hbm,
                 out_l, out_r, src_v, dst_v, rows_v, tmp_v, a_v, b2_v,
                 acc, table_sp, sem):
    c = lax.axis_index("c")
    s = lax.axis_index("s")
    rsl = pl.ds(s * ROWS_PER_TILE, ROWS_PER_TILE)
    pltpu.sync_copy(src_hbm.at[s], src_v)
    pltpu.sync_copy(dst_hbm.at[s], dst_v)
    pltpu.sync_copy(zeros_hbm, acc.at[rsl])
    pltpu.sync_copy(a_hbm.at[rsl], a_v)
    pltpu.sync_copy(b2_hbm.at[pl.ds(c * F_Q, F_Q)], b2_v)

    @pl.when(c == 0)
    def _():
        pltpu.sync_copy(p_l.at[rsl], table_sp.at[rsl])

    @pl.when(c == 1)
    def _():
        pltpu.sync_copy(p_r.at[rsl], table_sp.at[rsl])

    plsc.subcore_barrier()

    def body(j, carry):
        pltpu.async_copy(table_sp.at[src_v.at[j]], rows_v, sem).wait()
        pltpu.sync_copy(rows_v, acc.at[dst_v.at[j]], add=True)
        return carry

    lax.fori_loop(0, _CHUNKS_F, body, 0)
    plsc.subcore_barrier()
    pltpu.sync_copy(acc.at[rsl], tmp_v)
    b2a = b2_v[pl.ds(0, 16)]
    b2b = b2_v[pl.ds(16, 16)]

    def rbody(i, carry):
        av = a_v[i, pl.ds(0, 16)]
        tmp_v[i, pl.ds(0, 16)] = tmp_v[i, pl.ds(0, 16)] * av + b2a
        tmp_v[i, pl.ds(16, 16)] = tmp_v[i, pl.ds(16, 16)] * av + b2b
        return carry

    lax.fori_loop(0, ROWS_PER_TILE, rbody, 0)

    @pl.when(c == 0)
    def _():
        pltpu.sync_copy(tmp_v, out_l.at[rsl])

    @pl.when(c == 1)
    def _():
        pltpu.sync_copy(tmp_v, out_r.at[rsl])


_RB = 1280


def _prescale_body(deg_ref, x_ref, a_ref, q0_ref, q1_ref, q2_ref, q3_ref):
    deg = jnp.sum(deg_ref[...], axis=0)
    a = 1.0 / jnp.sqrt(jnp.maximum(deg, 1.0))
    a2 = a[:, None]
    a_ref[...] = a2
    xs = x_ref[...] * a2
    q0_ref[...] = xs[:, 0 * F_Q4:1 * F_Q4]
    q1_ref[...] = xs[:, 1 * F_Q4:2 * F_Q4]
    q2_ref[...] = xs[:, 2 * F_Q4:3 * F_Q4]
    q3_ref[...] = xs[:, 3 * F_Q4:4 * F_Q4]


def _dense_body(v0_ref, v1_ref, v2_ref, v3_ref, a_ref, w1_ref, b1_ref,
                w2_ref, pl_ref, pr_ref):
    a2 = a_ref[...]
    v1 = jnp.concatenate([v0_ref[...], v1_ref[...], v2_ref[...],
                          v3_ref[...]], axis=1)
    hp = jnp.dot(v1 * a2, w1_ref[...],
                 preferred_element_type=jnp.float32,
                 precision=lax.Precision.HIGHEST)
    h = jnp.maximum(hp + b1_ref[...], 0.0)
    p = jnp.dot(h, w2_ref[...],
                preferred_element_type=jnp.float32,
                precision=lax.Precision.HIGHEST) * a2
    pl_ref[...] = p[:, :F_Q]
    pr_ref[...] = p[:, F_Q:]


def _row_spec(width):
    return pl.BlockSpec((_RB, width), lambda i: (i, 0))


def _full_spec(shape):
    nd = len(shape)
    return pl.BlockSpec(shape, lambda i, _n=nd: (0,) * _n)


_prescale = pl.pallas_call(
    _prescale_body,
    grid=(N_PAD // _RB,),
    in_specs=[pl.BlockSpec((NC * NS, _RB), lambda i: (0, i)),
              _row_spec(F_IN)],
    out_specs=(_row_spec(1),) + tuple(_row_spec(F_Q4) for _ in range(4)),
    out_shape=(jax.ShapeDtypeStruct((N_PAD, 1), jnp.float32),)
    + tuple(jax.ShapeDtypeStruct((N_PAD, F_Q4), jnp.float32)
            for _ in range(4)),
)

_dense = pl.pallas_call(
    _dense_body,
    grid=(N_PAD // _RB,),
    in_specs=[_row_spec(F_Q4), _row_spec(F_Q4), _row_spec(F_Q4),
              _row_spec(F_Q4), _row_spec(1),
              _full_spec((F_IN, F_HID)),
              _full_spec((1, F_HID)), _full_spec((F_HID, F_OUT))],
    out_specs=(_row_spec(F_Q), _row_spec(F_Q)),
    out_shape=(jax.ShapeDtypeStruct((N_PAD, F_Q), jnp.float32),
               jax.ShapeDtypeStruct((N_PAD, F_Q), jnp.float32)),
)


def kernel(x, edge_index, edge_weight, W1, b1, W2, b2):
    del edge_weight
    src = edge_index[0]
    dst = edge_index[1]
    pad = E_PAD - N_EDGES
    src_p = jnp.concatenate([src, jnp.full((pad,), N_NODES, jnp.int32)])
    dst_p = jnp.concatenate([dst, jnp.full((pad,), N_NODES, jnp.int32)])
    src16 = src_p.reshape(NS, _CHUNKS_F, CH)
    dst16 = dst_p.reshape(NS, _CHUNKS_F, CH)
    x_pad = jnp.zeros((N_PAD, F_IN), jnp.float32).at[:N_NODES].set(x)

    z_wide = jnp.zeros((ROWS_PER_TILE, F_Q4), jnp.float32)
    z_nar = jnp.zeros((ROWS_PER_TILE, F_Q), jnp.float32)

    deg_parts = _deg_kernel(dst_p.reshape(NC * NS, _E_TILE))
    a, xq0, xq1, xq2, xq3 = _prescale(deg_parts, x_pad)
    v0, v1, v2, v3 = _spmm_wide(xq0, xq1, xq2, xq3, src16, dst16, z_wide)
    p_l, p_r = _dense(v0, v1, v2, v3, a, W1,
                      b1.reshape(1, F_HID), W2)
    o_l, o_r = _spmm_narrow(p_l, p_r, src16, dst16, z_nar,
                            jnp.broadcast_to(a, (N_PAD, 16)), b2)
    return jnp.concatenate([o_l, o_r], axis=1)[:N_NODES]
